# Initial kernel scaffold; baseline (speedup 1.0000x reference)
import functools

import jax
import jax.numpy as jnp
from jax import lax
from jax.experimental import pallas as pl
from jax.experimental.pallas import tpu as pltpu
from jax.experimental.pallas import tpu_sc as plsc

N = 10000
D = 256
H = 32
C = 16

NC = 2
NS = 16
NW = NC * NS
CHUNK = 128
NP = 10240
RPS = NP // NS

_mesh = plsc.VectorSubcoreMesh(core_axis_name="c", subcore_axis_name="s")


def _sc_degree(dst_pad, ones_c, zeros_1):
    ep = dst_pad.shape[0]
    epw = ep // NW
    chpw = epw // CHUNK

    @functools.partial(
        pl.kernel,
        mesh=_mesh,
        out_type=jax.ShapeDtypeStruct((NC, NP), jnp.float32),
        scratch_types=[
            pltpu.VMEM((CHUNK,), jnp.int32),
            pltpu.VMEM((CHUNK,), jnp.float32),
            pltpu.VMEM_SHARED((NP,), jnp.float32),
        ],
    )
    def k(dst_hbm, ones_hbm, z_hbm, out_hbm, idxv, onesv, acc):
        c = lax.axis_index("c")
        s = lax.axis_index("s")
        pltpu.sync_copy(ones_hbm, onesv)

        @pl.when(s == 0)
        def _():
            pltpu.sync_copy(z_hbm, acc)

        plsc.subcore_barrier()
        base0 = (c * NS + s) * epw

        @pl.loop(0, chpw)
        def _(i):
            base = base0 + i * CHUNK
            pltpu.sync_copy(dst_hbm.at[pl.ds(base, CHUNK)], idxv)
            pltpu.sync_copy(onesv, acc.at[idxv], add=True)

        plsc.subcore_barrier()
        pltpu.sync_copy(acc.at[pl.ds(s * RPS, RPS)],
                        out_hbm.at[c, pl.ds(s * RPS, RPS)])

    return k(dst_pad, ones_c, zeros_1)


def _sc_spmm(h, src_pad, dst_pad, zeros_h):
    ep = src_pad.shape[0]
    epw = ep // NW
    chpw = epw // CHUNK

    @functools.partial(
        pl.kernel,
        mesh=_mesh,
        out_type=jax.ShapeDtypeStruct((NC, NP, H), jnp.float32),
        scratch_types=[
            pltpu.VMEM((CHUNK,), jnp.int32),
            pltpu.VMEM((CHUNK,), jnp.int32),
            pltpu.VMEM((CHUNK, H), jnp.float32),
            pltpu.VMEM_SHARED((NP, H), jnp.float32),
        ],
    )
    def k(h_hbm, src_hbm, dst_hbm, z_hbm, out_hbm, srcv, dstv, rows, acc):
        c = lax.axis_index("c")
        s = lax.axis_index("s")

        @pl.when(s == 0)
        def _():
            pltpu.sync_copy(z_hbm, acc)

        plsc.subcore_barrier()
        base0 = (c * NS + s) * epw

        @pl.loop(0, chpw)
        def _(i):
            base = base0 + i * CHUNK
            pltpu.sync_copy(src_hbm.at[pl.ds(base, CHUNK)], srcv)
            pltpu.sync_copy(dst_hbm.at[pl.ds(base, CHUNK)], dstv)
            pltpu.sync_copy(h_hbm.at[srcv], rows)
            pltpu.sync_copy(rows, acc.at[dstv], add=True)

        plsc.subcore_barrier()
        pltpu.sync_copy(acc.at[pl.ds(s * RPS, RPS)],
                        out_hbm.at[c, pl.ds(s * RPS, RPS)])

    return k(h, src_pad, dst_pad, zeros_h)


def _tc_prelayer(x, W_hidden, d0, d1):
    BM = 500

    def body(x_ref, w_ref, d0_ref, d1_ref, h1_ref, dinv_ref):
        deg = d0_ref[...] + d1_ref[...] + 1.0
        dinv = lax.rsqrt(deg)
        xw = jnp.dot(x_ref[...], w_ref[...], preferred_element_type=jnp.float32)
        h1_ref[...] = xw * dinv
        dinv_ref[...] = dinv

    return pl.pallas_call(
        body,
        grid=(N // BM,),
        in_specs=[
            pl.BlockSpec((BM, D), lambda i: (i, 0)),
            pl.BlockSpec((D, H), lambda i: (0, 0)),
            pl.BlockSpec((BM, 1), lambda i: (i, 0)),
            pl.BlockSpec((BM, 1), lambda i: (i, 0)),
        ],
        out_specs=[
            pl.BlockSpec((BM, H), lambda i: (i, 0)),
            pl.BlockSpec((BM, 1), lambda i: (i, 0)),
        ],
        out_shape=[
            jax.ShapeDtypeStruct((N, H), jnp.float32),
            jax.ShapeDtypeStruct((N, 1), jnp.float32),
        ],
    )(x, W_hidden, d0, d1)


def _tc_midlayer(s10, s11, h1, dinv, Wcat):
    BM = 500

    def body(s0_ref, s1_ref, h1_ref, dinv_ref, w_ref, h2_ref):
        hidden = (s0_ref[...] + s1_ref[...] + h1_ref[...]) * dinv_ref[...]
        hidden = jnp.maximum(hidden, 0.0)
        hw = jnp.dot(hidden, w_ref[...], preferred_element_type=jnp.float32)
        h2_ref[...] = hw * dinv_ref[...]

    return pl.pallas_call(
        body,
        grid=(N // BM,),
        in_specs=[
            pl.BlockSpec((BM, H), lambda i: (i, 0)),
            pl.BlockSpec((BM, H), lambda i: (i, 0)),
            pl.BlockSpec((BM, H), lambda i: (i, 0)),
            pl.BlockSpec((BM, 1), lambda i: (i, 0)),
            pl.BlockSpec((H, H), lambda i: (0, 0)),
        ],
        out_specs=pl.BlockSpec((BM, H), lambda i: (i, 0)),
        out_shape=jax.ShapeDtypeStruct((N, H), jnp.float32),
    )(s10, s11, h1, dinv, Wcat)


def _tc_encode(s20, s21, h2, dinv, eps):
    BM = 500

    def body(s0_ref, s1_ref, h2_ref, dinv_ref, eps_ref, enc_ref):
        mls = (s0_ref[...] + s1_ref[...] + h2_ref[...]) * dinv_ref[...]
        means = mls[:, :C]
        std2 = jnp.exp(mls[:, C:])
        enc_ref[...] = means + std2 * (means + eps_ref[...])

    return pl.pallas_call(
        body,
        grid=(N // BM,),
        in_specs=[
            pl.BlockSpec((BM, H), lambda i: (i, 0)),
            pl.BlockSpec((BM, H), lambda i: (i, 0)),
            pl.BlockSpec((BM, H), lambda i: (i, 0)),
            pl.BlockSpec((BM, 1), lambda i: (i, 0)),
            pl.BlockSpec((BM, C), lambda i: (i, 0)),
        ],
        out_specs=pl.BlockSpec((BM, C), lambda i: (i, 0)),
        out_shape=jax.ShapeDtypeStruct((N, C), jnp.float32),
    )(s20, s21, h2, dinv, eps)


def _tc_decoder(enc):
    BM = 500

    def body(a_ref, b_ref, o_ref):
        prod = lax.dot_general(a_ref[...], b_ref[...],
                               (((1,), (1,)), ((), ())),
                               preferred_element_type=jnp.float32)
        o_ref[...] = jax.nn.sigmoid(prod)

    return pl.pallas_call(
        body,
        grid=(N // BM,),
        in_specs=[
            pl.BlockSpec((BM, C), lambda i: (i, 0)),
            pl.BlockSpec((N, C), lambda i: (0, 0)),
        ],
        out_specs=pl.BlockSpec((BM, N), lambda i: (i, 0)),
        out_shape=jax.ShapeDtypeStruct((N, N), jnp.float32),
    )(enc, enc)


def kernel(x, edge_index, W_hidden, W_means, W_logstd2):
    src = edge_index[0]
    dst = edge_index[1]
    e = src.shape[0]
    epw = pl.cdiv(e, NW * CHUNK) * CHUNK
    ep = epw * NW
    pad = ep - e
    src_p = jnp.concatenate([src, jnp.zeros((pad,), jnp.int32)])
    dst_p = jnp.concatenate([dst, jnp.full((pad,), N, jnp.int32)])

    ones_c = jnp.ones((CHUNK,), jnp.float32)
    zeros_1 = jnp.zeros((NP,), jnp.float32)
    zeros_h = jnp.zeros((NP, H), jnp.float32)

    degp = _sc_degree(dst_p, ones_c, zeros_1)
    d0 = degp[0, :N, None]
    d1 = degp[1, :N, None]
    h1, dinv = _tc_prelayer(x, W_hidden, d0, d1)

    s1 = _sc_spmm(h1, src_p, dst_p, zeros_h)
    Wcat = jnp.concatenate([W_means, W_logstd2], axis=1)
    h2 = _tc_midlayer(s1[0, :N], s1[1, :N], h1, dinv, Wcat)

    s2 = _sc_spmm(h2, src_p, dst_p, zeros_h)
    eps = jax.random.normal(jax.random.key(42), (N, C), jnp.float32)
    enc = _tc_encode(s2[0, :N], s2[1, :N], h2, dinv, eps)

    return _tc_decoder(enc)


# speedup vs baseline: 10.4507x; 10.4507x over previous
import functools

import jax
import jax.numpy as jnp
from jax import lax
from jax.experimental import pallas as pl
from jax.experimental.pallas import tpu as pltpu
from jax.experimental.pallas import tpu_sc as plsc

N = 10000
D = 256
H = 32
C = 16

NC = 2
NS = 16
NW = NC * NS
CHUNK = 128
NP = 10240
RPS = NP // NS

_mesh = plsc.VectorSubcoreMesh(core_axis_name="c", subcore_axis_name="s")


def _sc_degree(dst_pad, ones_c, zeros_1):
    ep = dst_pad.shape[0]
    epw = ep // NW
    chpw = epw // CHUNK

    @functools.partial(
        pl.kernel,
        mesh=_mesh,
        out_type=jax.ShapeDtypeStruct((NC, NP), jnp.float32),
        scratch_types=[
            pltpu.VMEM((CHUNK,), jnp.int32),
            pltpu.VMEM((CHUNK,), jnp.float32),
            pltpu.VMEM_SHARED((NP,), jnp.float32),
        ],
        compiler_params=pltpu.CompilerParams(use_tc_tiling_on_sc=False),
    )
    def k(dst_hbm, ones_hbm, z_hbm, out_hbm, idxv, onesv, acc):
        c = lax.axis_index("c")
        s = lax.axis_index("s")
        pltpu.sync_copy(ones_hbm, onesv)

        @pl.when(s == 0)
        def _():
            pltpu.sync_copy(z_hbm, acc)

        plsc.subcore_barrier()
        base0 = (c * NS + s) * epw

        @pl.loop(0, chpw)
        def _(i):
            base = base0 + i * CHUNK
            pltpu.sync_copy(dst_hbm.at[pl.ds(base, CHUNK)], idxv)
            pltpu.sync_copy(onesv, acc.at[idxv], add=True)

        plsc.subcore_barrier()
        pltpu.sync_copy(acc.at[pl.ds(s * RPS, RPS)],
                        out_hbm.at[c, pl.ds(s * RPS, RPS)])

    return k(dst_pad, ones_c, zeros_1)


def _sc_spmm(h, src_pad, dst_pad, zeros_h):
    ep = src_pad.shape[0]
    epw = ep // NW
    chpw = epw // CHUNK

    @functools.partial(
        pl.kernel,
        mesh=_mesh,
        out_type=jax.ShapeDtypeStruct((NC, NP, H), jnp.float32),
        scratch_types=[
            pltpu.VMEM((CHUNK,), jnp.int32),
            pltpu.VMEM((CHUNK,), jnp.int32),
            pltpu.VMEM((CHUNK, H), jnp.float32),
            pltpu.VMEM_SHARED((NP, H), jnp.float32),
        ],
        compiler_params=pltpu.CompilerParams(use_tc_tiling_on_sc=False),
    )
    def k(h_hbm, src_hbm, dst_hbm, z_hbm, out_hbm, srcv, dstv, rows, acc):
        c = lax.axis_index("c")
        s = lax.axis_index("s")

        @pl.when(s == 0)
        def _():
            pltpu.sync_copy(z_hbm, acc)

        plsc.subcore_barrier()
        base0 = (c * NS + s) * epw

        @pl.loop(0, chpw)
        def _(i):
            base = base0 + i * CHUNK
            pltpu.sync_copy(src_hbm.at[pl.ds(base, CHUNK)], srcv)
            pltpu.sync_copy(dst_hbm.at[pl.ds(base, CHUNK)], dstv)
            pltpu.sync_copy(h_hbm.at[srcv], rows)
            pltpu.sync_copy(rows, acc.at[dstv], add=True)

        plsc.subcore_barrier()
        pltpu.sync_copy(acc.at[pl.ds(s * RPS, RPS)],
                        out_hbm.at[c, pl.ds(s * RPS, RPS)])

    return k(h, src_pad, dst_pad, zeros_h)


def _tc_prelayer(x, W_hidden, d0, d1):
    BM = 400

    def body(x_ref, w_ref, d0_ref, d1_ref, h1_ref, dinv_ref):
        deg = d0_ref[...] + d1_ref[...] + 1.0
        dinv = lax.rsqrt(deg)
        xw = jnp.dot(x_ref[...], w_ref[...], preferred_element_type=jnp.float32)
        h1_ref[...] = xw * dinv
        dinv_ref[...] = dinv

    return pl.pallas_call(
        body,
        grid=(N // BM,),
        in_specs=[
            pl.BlockSpec((BM, D), lambda i: (i, 0)),
            pl.BlockSpec((D, H), lambda i: (0, 0)),
            pl.BlockSpec((BM, 1), lambda i: (i, 0)),
            pl.BlockSpec((BM, 1), lambda i: (i, 0)),
        ],
        out_specs=[
            pl.BlockSpec((BM, H), lambda i: (i, 0)),
            pl.BlockSpec((BM, 1), lambda i: (i, 0)),
        ],
        out_shape=[
            jax.ShapeDtypeStruct((N, H), jnp.float32),
            jax.ShapeDtypeStruct((N, 1), jnp.float32),
        ],
    )(x, W_hidden, d0, d1)


def _tc_midlayer(s10, s11, h1, dinv, Wcat):
    BM = 400

    def body(s0_ref, s1_ref, h1_ref, dinv_ref, w_ref, h2_ref):
        hidden = (s0_ref[...] + s1_ref[...] + h1_ref[...]) * dinv_ref[...]
        hidden = jnp.maximum(hidden, 0.0)
        hw = jnp.dot(hidden, w_ref[...], preferred_element_type=jnp.float32)
        h2_ref[...] = hw * dinv_ref[...]

    return pl.pallas_call(
        body,
        grid=(N // BM,),
        in_specs=[
            pl.BlockSpec((BM, H), lambda i: (i, 0)),
            pl.BlockSpec((BM, H), lambda i: (i, 0)),
            pl.BlockSpec((BM, H), lambda i: (i, 0)),
            pl.BlockSpec((BM, 1), lambda i: (i, 0)),
            pl.BlockSpec((H, H), lambda i: (0, 0)),
        ],
        out_specs=pl.BlockSpec((BM, H), lambda i: (i, 0)),
        out_shape=jax.ShapeDtypeStruct((N, H), jnp.float32),
    )(s10, s11, h1, dinv, Wcat)


def _tc_encode(s20, s21, h2, dinv, eps):
    BM = 400

    def body(s0_ref, s1_ref, h2_ref, dinv_ref, eps_ref, enc_ref):
        mls = (s0_ref[...] + s1_ref[...] + h2_ref[...]) * dinv_ref[...]
        means = mls[:, :C]
        std2 = jnp.exp(mls[:, C:])
        enc_ref[...] = means + std2 * (means + eps_ref[...])

    return pl.pallas_call(
        body,
        grid=(N // BM,),
        in_specs=[
            pl.BlockSpec((BM, H), lambda i: (i, 0)),
            pl.BlockSpec((BM, H), lambda i: (i, 0)),
            pl.BlockSpec((BM, H), lambda i: (i, 0)),
            pl.BlockSpec((BM, 1), lambda i: (i, 0)),
            pl.BlockSpec((BM, C), lambda i: (i, 0)),
        ],
        out_specs=pl.BlockSpec((BM, C), lambda i: (i, 0)),
        out_shape=jax.ShapeDtypeStruct((N, C), jnp.float32),
    )(s20, s21, h2, dinv, eps)


def _tc_decoder(enc):
    BM = 400

    def body(a_ref, b_ref, o_ref):
        prod = lax.dot_general(a_ref[...], b_ref[...],
                               (((1,), (1,)), ((), ())),
                               preferred_element_type=jnp.float32)
        o_ref[...] = jax.nn.sigmoid(prod)

    return pl.pallas_call(
        body,
        grid=(N // BM,),
        in_specs=[
            pl.BlockSpec((BM, C), lambda i: (i, 0)),
            pl.BlockSpec((N, C), lambda i: (0, 0)),
        ],
        out_specs=pl.BlockSpec((BM, N), lambda i: (i, 0)),
        out_shape=jax.ShapeDtypeStruct((N, N), jnp.float32),
    )(enc, enc)


def kernel(x, edge_index, W_hidden, W_means, W_logstd2):
    src = edge_index[0]
    dst = edge_index[1]
    e = src.shape[0]
    epw = pl.cdiv(e, NW * CHUNK) * CHUNK
    ep = epw * NW
    pad = ep - e
    src_p = jnp.concatenate([src, jnp.zeros((pad,), jnp.int32)])
    dst_p = jnp.concatenate([dst, jnp.full((pad,), N, jnp.int32)])

    ones_c = jnp.ones((CHUNK,), jnp.float32)
    zeros_1 = jnp.zeros((NP,), jnp.float32)
    zeros_h = jnp.zeros((NP, H), jnp.float32)

    degp = _sc_degree(dst_p, ones_c, zeros_1)
    d0 = degp[0, :N, None]
    d1 = degp[1, :N, None]
    h1, dinv = _tc_prelayer(x, W_hidden, d0, d1)

    s1 = _sc_spmm(h1, src_p, dst_p, zeros_h)
    Wcat = jnp.concatenate([W_means, W_logstd2], axis=1)
    h2 = _tc_midlayer(s1[0, :N], s1[1, :N], h1, dinv, Wcat)

    s2 = _sc_spmm(h2, src_p, dst_p, zeros_h)
    eps = jax.random.normal(jax.random.key(42), (N, C), jnp.float32)
    enc = _tc_encode(s2[0, :N], s2[1, :N], h2, dinv, eps)

    return _tc_decoder(enc)


# speedup vs baseline: 13.2213x; 1.2651x over previous
import functools

import jax
import jax.numpy as jnp
from jax import lax
from jax.experimental import pallas as pl
from jax.experimental.pallas import tpu as pltpu
from jax.experimental.pallas import tpu_sc as plsc

N = 10000
D = 256
H = 32
C = 16

NC = 2
NS = 16
NW = NC * NS
CHUNK = 128
NP = 10240
RPS = NP // NS
K = 4
KD = 8

_mesh = plsc.VectorSubcoreMesh(core_axis_name="c", subcore_axis_name="s")
_sc_params = pltpu.CompilerParams(use_tc_tiling_on_sc=False)


def _sc_degree(dst3, ones_c, zeros_1):
    chpw = dst3.shape[1]

    @functools.partial(
        pl.kernel,
        mesh=_mesh,
        out_type=jax.ShapeDtypeStruct((NC, NP), jnp.float32),
        scratch_types=[
            pltpu.VMEM((chpw, CHUNK), jnp.int32),
            pltpu.VMEM((CHUNK,), jnp.float32),
            pltpu.VMEM_SHARED((NP,), jnp.float32),
            pltpu.SemaphoreType.DMA,
        ],
        compiler_params=_sc_params,
    )
    def k(dst_hbm, ones_hbm, z_hbm, out_hbm, dstv, onesv, acc, sem):
        c = lax.axis_index("c")
        s = lax.axis_index("s")
        w = c * NS + s
        pltpu.sync_copy(dst_hbm.at[w], dstv)
        pltpu.sync_copy(ones_hbm, onesv)

        @pl.when(s == 0)
        def _():
            pltpu.sync_copy(z_hbm, acc)

        plsc.subcore_barrier()

        @pl.loop(0, chpw, step=KD)
        def _(j):
            ds = [pltpu.async_copy(onesv, acc.at[dstv.at[j + b]], sem, add=True)
                  for b in range(KD)]
            for d in ds:
                d.wait()

        plsc.subcore_barrier()
        pltpu.sync_copy(acc.at[pl.ds(s * RPS, RPS)],
                        out_hbm.at[c, pl.ds(s * RPS, RPS)])

    return k(dst3, ones_c, zeros_1)


def _sc_spmm(h, src3, dst3, zeros_h):
    chpw = src3.shape[1]
    nsup = chpw // K

    @functools.partial(
        pl.kernel,
        mesh=_mesh,
        out_type=jax.ShapeDtypeStruct((NC, NP, H), jnp.float32),
        scratch_types=[
            pltpu.VMEM((chpw, CHUNK), jnp.int32),
            pltpu.VMEM((chpw, CHUNK), jnp.int32),
            pltpu.VMEM((2, K, CHUNK, H), jnp.float32),
            pltpu.VMEM_SHARED((NP, H), jnp.float32),
            pltpu.SemaphoreType.DMA((2,)),
            pltpu.SemaphoreType.DMA((2,)),
        ],
        compiler_params=_sc_params,
    )
    def k(h_hbm, src_hbm, dst_hbm, z_hbm, out_hbm,
          srcv, dstv, rows, acc, gsem, ssem):
        c = lax.axis_index("c")
        s = lax.axis_index("s")
        w = c * NS + s
        pltpu.sync_copy(src_hbm.at[w], srcv)
        pltpu.sync_copy(dst_hbm.at[w], dstv)

        def fire_g(sup, half):
            for b in range(K):
                pltpu.async_copy(h_hbm.at[srcv.at[sup * K + b]],
                                 rows.at[half, b], gsem.at[half])

        def drain_g(half):
            for b in range(K):
                pltpu.make_async_copy(h_hbm.at[srcv.at[0]],
                                      rows.at[half, b], gsem.at[half]).wait()

        def fire_s(sup, half):
            for b in range(K):
                pltpu.async_copy(rows.at[half, b],
                                 acc.at[dstv.at[sup * K + b]],
                                 ssem.at[half], add=True)

        def drain_s(half):
            for b in range(K):
                pltpu.make_async_copy(rows.at[half, b],
                                      acc.at[dstv.at[0]],
                                      ssem.at[half]).wait()

        fire_g(0, 0)

        @pl.when(s == 0)
        def _():
            pltpu.sync_copy(z_hbm, acc)

        plsc.subcore_barrier()

        @pl.loop(0, nsup, step=2)
        def _(sp):
            for half in (0, 1):
                sup = sp + half
                other = 1 - half
                drain_g(half)

                @pl.when(sup >= 1)
                def _(other=other):
                    drain_s(other)

                @pl.when(sup + 1 < nsup)
                def _(sup=sup, other=other):
                    fire_g(sup + 1, other)

                fire_s(sup, half)

        drain_s((nsup - 1) % 2)
        plsc.subcore_barrier()
        pltpu.sync_copy(acc.at[pl.ds(s * RPS, RPS)],
                        out_hbm.at[c, pl.ds(s * RPS, RPS)])

    return k(h, src3, dst3, zeros_h)


def _tc_xw(x, W_hidden):
    BM = 400

    def body(x_ref, w_ref, xw_ref):
        xw_ref[...] = jnp.dot(x_ref[...], w_ref[...],
                              preferred_element_type=jnp.float32)

    return pl.pallas_call(
        body,
        grid=(N // BM,),
        in_specs=[
            pl.BlockSpec((BM, D), lambda i: (i, 0)),
            pl.BlockSpec((D, H), lambda i: (0, 0)),
        ],
        out_specs=pl.BlockSpec((BM, H), lambda i: (i, 0)),
        out_shape=jax.ShapeDtypeStruct((N, H), jnp.float32),
    )(x, W_hidden)


def _tc_scale(xw, d0, d1):
    BM = 400

    def body(xw_ref, d0_ref, d1_ref, h1_ref, dinv_ref):
        dinv = lax.rsqrt(d0_ref[...] + d1_ref[...] + 1.0)
        h1_ref[...] = xw_ref[...] * dinv
        dinv_ref[...] = dinv

    return pl.pallas_call(
        body,
        grid=(N // BM,),
        in_specs=[
            pl.BlockSpec((BM, H), lambda i: (i, 0)),
            pl.BlockSpec((BM, 1), lambda i: (i, 0)),
            pl.BlockSpec((BM, 1), lambda i: (i, 0)),
        ],
        out_specs=[
            pl.BlockSpec((BM, H), lambda i: (i, 0)),
            pl.BlockSpec((BM, 1), lambda i: (i, 0)),
        ],
        out_shape=[
            jax.ShapeDtypeStruct((N, H), jnp.float32),
            jax.ShapeDtypeStruct((N, 1), jnp.float32),
        ],
    )(xw, d0, d1)


def _tc_midlayer(s10, s11, h1, dinv, Wcat):
    BM = 400

    def body(s0_ref, s1_ref, h1_ref, dinv_ref, w_ref, h2_ref):
        hidden = (s0_ref[...] + s1_ref[...] + h1_ref[...]) * dinv_ref[...]
        hidden = jnp.maximum(hidden, 0.0)
        hw = jnp.dot(hidden, w_ref[...], preferred_element_type=jnp.float32)
        h2_ref[...] = hw * dinv_ref[...]

    return pl.pallas_call(
        body,
        grid=(N // BM,),
        in_specs=[
            pl.BlockSpec((BM, H), lambda i: (i, 0)),
            pl.BlockSpec((BM, H), lambda i: (i, 0)),
            pl.BlockSpec((BM, H), lambda i: (i, 0)),
            pl.BlockSpec((BM, 1), lambda i: (i, 0)),
            pl.BlockSpec((H, H), lambda i: (0, 0)),
        ],
        out_specs=pl.BlockSpec((BM, H), lambda i: (i, 0)),
        out_shape=jax.ShapeDtypeStruct((N, H), jnp.float32),
    )(s10, s11, h1, dinv, Wcat)


def _tc_encode(s20, s21, h2, dinv, eps):
    BM = 400

    def body(s0_ref, s1_ref, h2_ref, dinv_ref, eps_ref, enc_ref):
        mls = (s0_ref[...] + s1_ref[...] + h2_ref[...]) * dinv_ref[...]
        means = mls[:, :C]
        std2 = jnp.exp(mls[:, C:])
        enc_ref[...] = means + std2 * (means + eps_ref[...])

    return pl.pallas_call(
        body,
        grid=(N // BM,),
        in_specs=[
            pl.BlockSpec((BM, H), lambda i: (i, 0)),
            pl.BlockSpec((BM, H), lambda i: (i, 0)),
            pl.BlockSpec((BM, H), lambda i: (i, 0)),
            pl.BlockSpec((BM, 1), lambda i: (i, 0)),
            pl.BlockSpec((BM, C), lambda i: (i, 0)),
        ],
        out_specs=pl.BlockSpec((BM, C), lambda i: (i, 0)),
        out_shape=jax.ShapeDtypeStruct((N, C), jnp.float32),
    )(s20, s21, h2, dinv, eps)


def _tc_decoder(enc):
    BM = 400

    def body(a_ref, b_ref, o_ref):
        prod = lax.dot_general(a_ref[...], b_ref[...],
                               (((1,), (1,)), ((), ())),
                               preferred_element_type=jnp.float32)
        o_ref[...] = jax.nn.sigmoid(prod)

    return pl.pallas_call(
        body,
        grid=(N // BM,),
        in_specs=[
            pl.BlockSpec((BM, C), lambda i: (i, 0)),
            pl.BlockSpec((N, C), lambda i: (0, 0)),
        ],
        out_specs=pl.BlockSpec((BM, N), lambda i: (i, 0)),
        out_shape=jax.ShapeDtypeStruct((N, N), jnp.float32),
    )(enc, enc)


def kernel(x, edge_index, W_hidden, W_means, W_logstd2):
    src = edge_index[0]
    dst = edge_index[1]
    e = src.shape[0]
    epw = pl.cdiv(e, NW * CHUNK) * CHUNK
    ep = epw * NW
    pad = ep - e
    chpw = epw // CHUNK
    src3 = jnp.concatenate([src, jnp.zeros((pad,), jnp.int32)])
    dst3 = jnp.concatenate([dst, jnp.full((pad,), N, jnp.int32)])
    src3 = src3.reshape(NW, chpw, CHUNK)
    dst3 = dst3.reshape(NW, chpw, CHUNK)

    ones_c = jnp.ones((CHUNK,), jnp.float32)
    zeros_1 = jnp.zeros((NP,), jnp.float32)
    zeros_h = jnp.zeros((NP, H), jnp.float32)

    degp = _sc_degree(dst3, ones_c, zeros_1)
    xw = _tc_xw(x, W_hidden)
    d0 = degp[0, :N, None]
    d1 = degp[1, :N, None]
    h1, dinv = _tc_scale(xw, d0, d1)

    s1 = _sc_spmm(h1, src3, dst3, zeros_h)
    Wcat = jnp.concatenate([W_means, W_logstd2], axis=1)
    h2 = _tc_midlayer(s1[0, :N], s1[1, :N], h1, dinv, Wcat)

    s2 = _sc_spmm(h2, src3, dst3, zeros_h)
    eps = jax.random.normal(jax.random.key(42), (N, C), jnp.float32)
    enc = _tc_encode(s2[0, :N], s2[1, :N], h2, dinv, eps)

    return _tc_decoder(enc)
